# Initial kernel scaffold; baseline (speedup 1.0000x reference)
import jax
import jax.numpy as jnp
from jax.experimental import pallas as pl


def kernel(previous_ast_nodes_encodings, new_cfg_nodes_encodings, key_indices, value_indices, W_g, b_g, W_c, b_c):
    raise NotImplementedError("write your pallas kernel here")


# speedup vs baseline: 6.2280x; 6.2280x over previous
import functools

import jax
import jax.numpy as jnp
from jax import lax
from jax.experimental import pallas as pl
from jax.experimental.pallas import tpu as pltpu
from jax.experimental.pallas import tpu_sc as plsc

N_AST = 200000
N_CFG = 65536
K = 131072
D = 128

NC = 2
NS = 16
NW = NC * NS
L = 16

KPW = K // NW
C1 = 128

S_SPLIT = 8
Q_SPLIT = 4
KPS = K // S_SPLIT
RPW = 6400
RQ = 8 * RPW
CC = 128
CR = 128
NPOS = RPW + 16

_mesh = plsc.VectorSubcoreMesh(core_axis_name="c", subcore_axis_name="s")
_sc_params = pltpu.CompilerParams(needs_layout_passes=False)


def _wid():
  return lax.axis_index("s") * NC + lax.axis_index("c")


def _gather_body(ast_hbm, cfg_hbm, ki_hbm, vi_hbm, gp_hbm, gu_hbm,
                 kiv, viv, bufp, bufu, sg0, sg1, sw0, sw1):
  wid = _wid()
  kbase = wid * KPW
  pltpu.sync_copy(ki_hbm.at[wid], kiv)
  pltpu.sync_copy(vi_hbm.at[wid], viv)
  sg = (sg0, sg1)
  sw = (sw0, sw1)
  nch = KPW // C1

  def g_issue(i, b):
    pltpu.async_copy(ast_hbm.at[kiv.at[i]], bufp.at[b], sg[b])
    pltpu.async_copy(cfg_hbm.at[viv.at[i]], bufu.at[b], sg[b])

  def g_wait(i, b):
    pltpu.make_async_copy(ast_hbm.at[kiv.at[i]], bufp.at[b], sg[b]).wait()
    pltpu.make_async_copy(cfg_hbm.at[viv.at[i]], bufu.at[b], sg[b]).wait()

  def w_issue(i, b):
    o = kbase + i * C1
    pltpu.async_copy(bufp.at[b], gp_hbm.at[pl.ds(o, C1)], sw[b])
    pltpu.async_copy(bufu.at[b], gu_hbm.at[pl.ds(o, C1)], sw[b])

  def w_wait(i, b):
    o = kbase + i * C1
    pltpu.make_async_copy(bufp.at[b], gp_hbm.at[pl.ds(o, C1)], sw[b]).wait()
    pltpu.make_async_copy(bufu.at[b], gu_hbm.at[pl.ds(o, C1)], sw[b]).wait()

  g_issue(0, 0)
  g_issue(1, 1)

  def outer(i2, carry):
    for b in range(2):
      i = i2 * 2 + b
      g_wait(i, b)
      w_issue(i, b)
      nxt = i + 2
      @pl.when(nxt < nch)
      def _():
        w_wait(i, b)
        g_issue(nxt, b)
    return carry

  lax.fori_loop(0, nch // 2, outer, 0)
  w_wait(nch - 2, 0)
  w_wait(nch - 1, 1)


@functools.partial(
    pl.kernel,
    out_type=(jax.ShapeDtypeStruct((K, D), jnp.float32),
              jax.ShapeDtypeStruct((K, D), jnp.float32)),
    mesh=_mesh,
    compiler_params=_sc_params,
    scratch_types=[
        pltpu.VMEM((KPW // C1, C1), jnp.int32),
        pltpu.VMEM((KPW // C1, C1), jnp.int32),
        pltpu.VMEM((2, C1, D), jnp.float32),
        pltpu.VMEM((2, C1, D), jnp.float32),
        pltpu.SemaphoreType.DMA,
        pltpu.SemaphoreType.DMA,
        pltpu.SemaphoreType.DMA,
        pltpu.SemaphoreType.DMA,
    ],
)
def _gather_call(*refs):
  _gather_body(*refs)


BK = 8192


def _mlp_body(gp_ref, gu_ref, wg1_ref, wg2_ref, bg_ref, wc_ref, bc_ref, out_ref):
  prev = gp_ref[...]
  upd = gu_ref[...]
  z = jnp.dot(prev, wg1_ref[...], preferred_element_type=jnp.float32)
  z = z + jnp.dot(upd, wg2_ref[...], preferred_element_type=jnp.float32)
  z = z + bg_ref[...]
  g = jax.nn.sigmoid(z)
  cand = jnp.dot(upd, wc_ref[...], preferred_element_type=jnp.float32) + bc_ref[...]
  cand = jnp.maximum(cand, 0.0)
  out_ref[...] = g * prev + (1.0 - g) * cand


_mlp_call = pl.pallas_call(
    _mlp_body,
    grid=(K // BK,),
    in_specs=[
        pl.BlockSpec((BK, D), lambda i: (i, 0)),
        pl.BlockSpec((BK, D), lambda i: (i, 0)),
        pl.BlockSpec((D, D), lambda i: (0, 0)),
        pl.BlockSpec((D, D), lambda i: (0, 0)),
        pl.BlockSpec((1, D), lambda i: (0, 0)),
        pl.BlockSpec((D, D), lambda i: (0, 0)),
        pl.BlockSpec((1, D), lambda i: (0, 0)),
    ],
    out_specs=pl.BlockSpec((BK, D), lambda i: (i, 0)),
    out_shape=jax.ShapeDtypeStruct((K, D), jnp.float32),
)


def _scan_body(ki_hbm, part_hbm, keys_v, lastpos_v):
  wid = _wid()
  s_idx = wid % S_SPLIT
  q_idx = wid // S_SPLIT
  rq0 = q_idx * RQ
  kofs = s_idx * KPS
  pltpu.sync_copy(ki_hbm.at[pl.ds(kofs, KPS)], keys_v)

  neg1 = jnp.full((L,), -1, jnp.int32)

  def init(i, c):
    lastpos_v[pl.ds(i * L, L)] = neg1
    return c

  lax.fori_loop(0, RQ // L, init, 0)

  lane = lax.iota(jnp.int32, L)

  big = jnp.full((L,), 0x7FFFFFFF, jnp.int32)
  lane_next = jnp.minimum(lane + 1, L - 1)

  def scan(i, c):
    keys = keys_v[pl.ds(i * L, L)]
    inr = (keys >= rq0) & (keys < rq0 + RQ)
    comp = jnp.where(inr, ((keys - rq0) << 4) | lane, big)
    cs = jnp.sort(comp)
    locs = cs >> 4
    nxt = locs.at[lane_next].get(mode="promise_in_bounds")
    valid = cs != big
    winner = valid & ((locs != nxt) | (lane == L - 1))
    kvec = (kofs + i * L) + (cs & (L - 1))
    loc_safe = jnp.where(winner, locs, 0)
    plsc.store_scatter(lastpos_v, [loc_safe], kvec, mask=winner)
    return c

  lax.fori_loop(0, KPS // L, scan, 0)
  pltpu.sync_copy(lastpos_v, part_hbm.at[wid])


@functools.partial(
    pl.kernel,
    out_type=jax.ShapeDtypeStruct((NW, RQ), jnp.int32),
    mesh=_mesh,
    compiler_params=_sc_params,
    scratch_types=[
        pltpu.VMEM((KPS,), jnp.int32),
        pltpu.VMEM((RQ,), jnp.int32),
    ],
)
def _scan_call(*refs):
  _scan_body(*refs)


def _scatter_body(ast_hbm, upd_hbm, part_hbm, out_hbm,
                  lp_v, mrg_v, posf_v, dstf_v, cbuf, rbuf, r16,
                  dstb0, dstb1, posb0, posb1, scr0, scr1, scw0, scw1,
                  ssg0, ssg1, ssw0, ssw1, st):
  wid = _wid()
  rbase = wid * RPW
  q_idx = wid // S_SPLIT
  off = (wid % S_SPLIT) * RPW
  lane = lax.iota(jnp.int32, L)
  is_last = wid == (NW - 1)
  nrows = jnp.where(is_last, N_AST - (NW - 1) * RPW, RPW)

  pltpu.sync_copy(part_hbm.at[q_idx * S_SPLIT, pl.ds(off, RPW)], lp_v)
  for s in range(1, S_SPLIT):
    pltpu.sync_copy(part_hbm.at[q_idx * S_SPLIT + s, pl.ds(off, RPW)], mrg_v)

    def mrg(i, c):
      sl = pl.ds(i * L, L)
      lp_v[sl] = jnp.maximum(lp_v[sl], mrg_v[sl])
      return c

    lax.fori_loop(0, RPW // L, mrg, 0)

  scr = (scr0, scr1)
  scw = (scw0, scw1)

  def c_rd_issue(i, b):
    pltpu.async_copy(ast_hbm.at[pl.ds(rbase + i * CC, CC)], cbuf.at[b], scr[b])

  def c_rd_wait(i, b):
    pltpu.make_async_copy(ast_hbm.at[pl.ds(rbase + i * CC, CC)], cbuf.at[b], scr[b]).wait()

  def c_wr_issue(i, b):
    pltpu.async_copy(cbuf.at[b], out_hbm.at[pl.ds(rbase + i * CC, CC)], scw[b])

  def c_wr_wait(i, b):
    pltpu.make_async_copy(cbuf.at[b], out_hbm.at[pl.ds(rbase + i * CC, CC)], scw[b]).wait()

  nfc = nrows // CC
  c_rd_issue(0, 0)
  c_rd_issue(1, 1)

  def couter(i2, carry):
    for b in range(2):
      i = i2 * 2 + b
      c_rd_wait(i, b)
      c_wr_issue(i, b)
      nxt = i + 2
      @pl.when(nxt < nfc)
      def _():
        c_wr_wait(i, b)
        c_rd_issue(nxt, b)
    return carry

  lax.fori_loop(0, nfc // 2, couter, 0)
  c_wr_wait(nfc - 2, 0)
  c_wr_wait(nfc - 1, 1)

  @pl.when(is_last)
  def _():
    t0 = rbase + nfc * CC
    pltpu.async_copy(ast_hbm.at[pl.ds(t0, 64)], cbuf.at[0, pl.ds(0, 64)], scr[0])
    pltpu.make_async_copy(ast_hbm.at[pl.ds(t0, 64)], cbuf.at[0, pl.ds(0, 64)], scr[0]).wait()
    pltpu.async_copy(cbuf.at[0, pl.ds(0, 64)], out_hbm.at[pl.ds(t0, 64)], scw[0])
    pltpu.make_async_copy(cbuf.at[0, pl.ds(0, 64)], out_hbm.at[pl.ds(t0, 64)], scw[0]).wait()

  def compact(i, cnt):
    lp = lp_v[pl.ds(i * L, L)]
    valid = lp >= 0
    dest = (rbase + i * L) + lane
    plsc.store_compressed(posf_v.at[pl.ds(cnt, L)], lp, mask=valid)
    plsc.store_compressed(dstf_v.at[pl.ds(cnt, L)], dest, mask=valid)
    pop = plsc.all_reduce_population_count(valid)
    return cnt + jnp.max(pop)

  count = lax.fori_loop(0, RPW // L, compact, jnp.int32(0))

  ssg = (ssg0, ssg1)
  ssw = (ssw0, ssw1)
  dstb = (dstb0, dstb1)

  posb = (posb0, posb1)

  def s_g_issue(i, b):
    for j in range(CR // L):
      posb[b][pl.ds(j * L, L)] = posf_v[pl.ds(i * CR + j * L, L)]
    pltpu.async_copy(upd_hbm.at[posb[b]], rbuf.at[b], ssg[b])

  def s_g_wait(i, b):
    pltpu.make_async_copy(upd_hbm.at[posb[b]], rbuf.at[b], ssg[b]).wait()

  def s_w_issue(i, b):
    for j in range(CR // L):
      dstb[b][pl.ds(j * L, L)] = dstf_v[pl.ds(i * CR + j * L, L)]
    pltpu.async_copy(rbuf.at[b], out_hbm.at[dstb[b]], ssw[b])

  def s_w_wait(b):
    pltpu.make_async_copy(rbuf.at[b], out_hbm.at[dstb[b]], ssw[b]).wait()

  nfull = count // CR

  @pl.when(nfull > 0)
  def _():
    s_g_issue(0, 0)

  def sloop(i, carry):
    even = (i % 2) == 0

    @pl.when((i + 1 < nfull) & (i >= 1))
    def _():
      @pl.when(even)
      def _():
        s_w_wait(1)
      @pl.when(jnp.logical_not(even))
      def _():
        s_w_wait(0)

    @pl.when(i + 1 < nfull)
    def _():
      @pl.when(even)
      def _():
        s_g_issue(i + 1, 1)
      @pl.when(jnp.logical_not(even))
      def _():
        s_g_issue(i + 1, 0)

    @pl.when(even)
    def _():
      s_g_wait(i, 0)
      s_w_issue(i, 0)
    @pl.when(jnp.logical_not(even))
    def _():
      s_g_wait(i, 1)
      s_w_issue(i, 1)
    return carry

  lax.fori_loop(0, nfull, sloop, 0)
  @pl.when(nfull == 1)
  def _():
    s_w_wait(0)
  @pl.when(nfull >= 2)
  def _():
    s_w_wait(0)
    s_w_wait(1)

  base16 = nfull * CR
  t16 = (count - base16) // L

  def tail16(j, carry):
    o = base16 + j * L
    pos16 = posf_v[pl.ds(o, L)]
    dst16 = dstf_v[pl.ds(o, L)]
    pltpu.async_copy(upd_hbm.at[pos16], r16, st).wait()
    pltpu.async_copy(r16, out_hbm.at[dst16], st).wait()
    return carry

  lax.fori_loop(0, t16, tail16, 0)

  rem = count - base16 - t16 * L

  @pl.when((rem > 0) & (count >= L))
  def _():
    o = count - L
    pos16 = posf_v[pl.ds(o, L)]
    dst16 = dstf_v[pl.ds(o, L)]
    pltpu.async_copy(upd_hbm.at[pos16], r16, st).wait()
    pltpu.async_copy(r16, out_hbm.at[dst16], st).wait()

  @pl.when((rem > 0) & (count < L))
  def _():
    posv = posf_v[pl.ds(0, L)]
    dstv = dstf_v[pl.ds(0, L)]
    valid = lane < count
    minv = jnp.full((L,), -2147483648, jnp.int32)
    p0 = jnp.max(jnp.where(lane == 0, posv, minv))
    d0 = jnp.max(jnp.where(lane == 0, dstv, minv))
    pos16 = jnp.where(valid, posv, p0)
    dst16 = jnp.where(valid, dstv, d0)
    pltpu.async_copy(upd_hbm.at[pos16], r16, st).wait()
    pltpu.async_copy(r16, out_hbm.at[dst16], st).wait()


@functools.partial(
    pl.kernel,
    out_type=jax.ShapeDtypeStruct((N_AST, D), jnp.float32),
    mesh=_mesh,
    compiler_params=_sc_params,
    scratch_types=[
        pltpu.VMEM((RPW,), jnp.int32),
        pltpu.VMEM((RPW,), jnp.int32),
        pltpu.VMEM((NPOS,), jnp.int32),
        pltpu.VMEM((NPOS,), jnp.int32),
        pltpu.VMEM((2, CC, D), jnp.float32),
        pltpu.VMEM((2, CR, D), jnp.float32),
        pltpu.VMEM((L, D), jnp.float32),
        pltpu.VMEM((CR,), jnp.int32),
        pltpu.VMEM((CR,), jnp.int32),
        pltpu.VMEM((CR,), jnp.int32),
        pltpu.VMEM((CR,), jnp.int32),
        pltpu.SemaphoreType.DMA,
        pltpu.SemaphoreType.DMA,
        pltpu.SemaphoreType.DMA,
        pltpu.SemaphoreType.DMA,
        pltpu.SemaphoreType.DMA,
        pltpu.SemaphoreType.DMA,
        pltpu.SemaphoreType.DMA,
        pltpu.SemaphoreType.DMA,
        pltpu.SemaphoreType.DMA,
    ],
)
def _scatter_call(*refs):
  _scatter_body(*refs)


def kernel(previous_ast_nodes_encodings, new_cfg_nodes_encodings,
           key_indices, value_indices, W_g, b_g, W_c, b_c):
  ki = key_indices.astype(jnp.int32)
  vi = value_indices.astype(jnp.int32)
  ki3 = ki.reshape(NW, KPW // C1, C1)
  vi3 = vi.reshape(NW, KPW // C1, C1)
  gp, gu = _gather_call(previous_ast_nodes_encodings, new_cfg_nodes_encodings,
                        ki3, vi3)
  upd = _mlp_call(gp, gu, W_g[:D], W_g[D:], b_g.reshape(1, D),
                  W_c, b_c.reshape(1, D))
  part = _scan_call(ki)
  out = _scatter_call(previous_ast_nodes_encodings, upd, part)
  return out


# speedup vs baseline: 6.8856x; 1.1056x over previous
import functools

import jax
import jax.numpy as jnp
from jax import lax
from jax.experimental import pallas as pl
from jax.experimental.pallas import tpu as pltpu
from jax.experimental.pallas import tpu_sc as plsc

N_AST = 200000
N_CFG = 65536
K = 131072
D = 128

NC = 2
NS = 16
NW = NC * NS
L = 16

KPW = K // NW
C1 = 128

S_SPLIT = 8
Q_SPLIT = 4
KPS = K // S_SPLIT
RPW = 6400
RQ = 8 * RPW
CC = 128
CR = 128
NPOS = RPW + 16

_mesh = plsc.VectorSubcoreMesh(core_axis_name="c", subcore_axis_name="s")
_sc_params = pltpu.CompilerParams(needs_layout_passes=False)


def _wid():
  return lax.axis_index("s") * NC + lax.axis_index("c")


def _gather_body(ast_hbm, cfg_hbm, ki_hbm, vi_hbm, gp_hbm, gu_hbm,
                 kiv, viv, bufp, bufu, sg0, sg1, sw0, sw1):
  wid = _wid()
  kbase = wid * KPW
  pltpu.sync_copy(ki_hbm.at[wid], kiv)
  pltpu.sync_copy(vi_hbm.at[wid], viv)
  sg = (sg0, sg1)
  sw = (sw0, sw1)
  nch = KPW // C1

  def g_issue(i, b):
    pltpu.async_copy(ast_hbm.at[kiv.at[i]], bufp.at[b], sg[b])
    pltpu.async_copy(cfg_hbm.at[viv.at[i]], bufu.at[b], sg[b])

  def g_wait(i, b):
    pltpu.make_async_copy(ast_hbm.at[kiv.at[i]], bufp.at[b], sg[b]).wait()
    pltpu.make_async_copy(cfg_hbm.at[viv.at[i]], bufu.at[b], sg[b]).wait()

  def w_issue(i, b):
    o = kbase + i * C1
    pltpu.async_copy(bufp.at[b], gp_hbm.at[pl.ds(o, C1)], sw[b])
    pltpu.async_copy(bufu.at[b], gu_hbm.at[pl.ds(o, C1)], sw[b])

  def w_wait(i, b):
    o = kbase + i * C1
    pltpu.make_async_copy(bufp.at[b], gp_hbm.at[pl.ds(o, C1)], sw[b]).wait()
    pltpu.make_async_copy(bufu.at[b], gu_hbm.at[pl.ds(o, C1)], sw[b]).wait()

  g_issue(0, 0)
  g_issue(1, 1)

  def outer(i2, carry):
    for b in range(2):
      i = i2 * 2 + b
      g_wait(i, b)
      w_issue(i, b)
      nxt = i + 2
      @pl.when(nxt < nch)
      def _():
        w_wait(i, b)
        g_issue(nxt, b)
    return carry

  lax.fori_loop(0, nch // 2, outer, 0)
  w_wait(nch - 2, 0)
  w_wait(nch - 1, 1)


@functools.partial(
    pl.kernel,
    out_type=(jax.ShapeDtypeStruct((K, D), jnp.float32),
              jax.ShapeDtypeStruct((K, D), jnp.float32)),
    mesh=_mesh,
    compiler_params=_sc_params,
    scratch_types=[
        pltpu.VMEM((KPW // C1, C1), jnp.int32),
        pltpu.VMEM((KPW // C1, C1), jnp.int32),
        pltpu.VMEM((2, C1, D), jnp.float32),
        pltpu.VMEM((2, C1, D), jnp.float32),
        pltpu.SemaphoreType.DMA,
        pltpu.SemaphoreType.DMA,
        pltpu.SemaphoreType.DMA,
        pltpu.SemaphoreType.DMA,
    ],
)
def _gather_call(*refs):
  _gather_body(*refs)


BK = 8192


def _mlp_body(gp_ref, gu_ref, wg1_ref, wg2_ref, bg_ref, wc_ref, bc_ref, out_ref):
  prev = gp_ref[...]
  upd = gu_ref[...]
  z = jnp.dot(prev, wg1_ref[...], preferred_element_type=jnp.float32)
  z = z + jnp.dot(upd, wg2_ref[...], preferred_element_type=jnp.float32)
  z = z + bg_ref[...]
  g = jax.nn.sigmoid(z)
  cand = jnp.dot(upd, wc_ref[...], preferred_element_type=jnp.float32) + bc_ref[...]
  cand = jnp.maximum(cand, 0.0)
  out_ref[...] = g * prev + (1.0 - g) * cand


_mlp_call = pl.pallas_call(
    _mlp_body,
    grid=(K // BK,),
    in_specs=[
        pl.BlockSpec((BK, D), lambda i: (i, 0)),
        pl.BlockSpec((BK, D), lambda i: (i, 0)),
        pl.BlockSpec((D, D), lambda i: (0, 0)),
        pl.BlockSpec((D, D), lambda i: (0, 0)),
        pl.BlockSpec((1, D), lambda i: (0, 0)),
        pl.BlockSpec((D, D), lambda i: (0, 0)),
        pl.BlockSpec((1, D), lambda i: (0, 0)),
    ],
    out_specs=pl.BlockSpec((BK, D), lambda i: (i, 0)),
    out_shape=jax.ShapeDtypeStruct((K, D), jnp.float32),
)


def _scan_body(ki_hbm, part_hbm, keys_v, lastpos_v):
  wid = _wid()
  s_idx = wid % S_SPLIT
  q_idx = wid // S_SPLIT
  rq0 = q_idx * RQ
  kofs = s_idx * KPS
  pltpu.sync_copy(ki_hbm.at[pl.ds(kofs, KPS)], keys_v)

  neg1 = jnp.full((L,), -1, jnp.int32)

  def init(i, c):
    lastpos_v[pl.ds(i * L, L)] = neg1
    return c

  lax.fori_loop(0, RQ // L, init, 0)

  lane = lax.iota(jnp.int32, L)

  big = jnp.full((L,), 0x7FFFFFFF, jnp.int32)
  lane_next = jnp.minimum(lane + 1, L - 1)

  def scan(i, c):
    keys = keys_v[pl.ds(i * L, L)]
    inr = (keys >= rq0) & (keys < rq0 + RQ)
    comp = jnp.where(inr, ((keys - rq0) << 4) | lane, big)
    cs = jnp.sort(comp)
    locs = cs >> 4
    nxt = locs.at[lane_next].get(mode="promise_in_bounds")
    valid = cs != big
    winner = valid & ((locs != nxt) | (lane == L - 1))
    kvec = (kofs + i * L) + (cs & (L - 1))
    loc_safe = jnp.where(winner, locs, 0)
    plsc.store_scatter(lastpos_v, [loc_safe], kvec, mask=winner)
    return c

  lax.fori_loop(0, KPS // L, scan, 0)
  pltpu.sync_copy(lastpos_v, part_hbm.at[wid])


@functools.partial(
    pl.kernel,
    out_type=jax.ShapeDtypeStruct((NW, RQ), jnp.int32),
    mesh=_mesh,
    compiler_params=_sc_params,
    scratch_types=[
        pltpu.VMEM((KPS,), jnp.int32),
        pltpu.VMEM((RQ,), jnp.int32),
    ],
)
def _scan_call(*refs):
  _scan_body(*refs)


def _scatter_body(ast_hbm, upd_hbm, part_hbm, out_hbm,
                  lp_v, mrg_v, posf_v, dstf_v, dst2_v, rbuf, r16,
                  posb0, posb1, dstb0, dstb1,
                  ssg0, ssg1, ssw0, ssw1, st):
  wid = _wid()
  rbase = wid * RPW
  q_idx = wid // S_SPLIT
  off = (wid % S_SPLIT) * RPW
  lane = lax.iota(jnp.int32, L)

  pltpu.sync_copy(part_hbm.at[q_idx * S_SPLIT, pl.ds(off, RPW)], lp_v)
  for s in range(1, S_SPLIT):
    pltpu.sync_copy(part_hbm.at[q_idx * S_SPLIT + s, pl.ds(off, RPW)], mrg_v)

    def mrg(i, c):
      sl = pl.ds(i * L, L)
      lp_v[sl] = jnp.maximum(lp_v[sl], mrg_v[sl])
      return c

    lax.fori_loop(0, RPW // L, mrg, 0)

  def compact(i, carry):
    cnt, cnt2 = carry
    lp = lp_v[pl.ds(i * L, L)]
    grow = (rbase + i * L) + lane
    w = lp >= 0
    nw = jnp.logical_not(w) & (grow < N_AST)
    plsc.store_compressed(posf_v.at[pl.ds(cnt, L)], lp, mask=w)
    plsc.store_compressed(dstf_v.at[pl.ds(cnt, L)], grow, mask=w)
    plsc.store_compressed(dst2_v.at[pl.ds(cnt2, L)], grow, mask=nw)
    cnt = cnt + jnp.max(plsc.all_reduce_population_count(w))
    cnt2 = cnt2 + jnp.max(plsc.all_reduce_population_count(nw))
    return (cnt, cnt2)

  count, count2 = lax.fori_loop(0, RPW // L, compact,
                                (jnp.int32(0), jnp.int32(0)))

  ssg = (ssg0, ssg1)
  ssw = (ssw0, ssw1)
  posb = (posb0, posb1)
  dstb = (dstb0, dstb1)
  minv = jnp.full((L,), -2147483648, jnp.int32)

  def stream(src_hbm, pos_ref, dst_ref, n):
    def g_issue(i, b):
      for j in range(CR // L):
        posb[b][pl.ds(j * L, L)] = pos_ref[pl.ds(i * CR + j * L, L)]
      pltpu.async_copy(src_hbm.at[posb[b]], rbuf.at[b], ssg[b])

    def g_wait(b):
      pltpu.make_async_copy(src_hbm.at[posb[b]], rbuf.at[b], ssg[b]).wait()

    def w_issue(i, b):
      for j in range(CR // L):
        dstb[b][pl.ds(j * L, L)] = dst_ref[pl.ds(i * CR + j * L, L)]
      pltpu.async_copy(rbuf.at[b], out_hbm.at[dstb[b]], ssw[b])

    def w_wait(b):
      pltpu.make_async_copy(rbuf.at[b], out_hbm.at[dstb[b]], ssw[b]).wait()

    nfull = n // CR

    @pl.when(nfull > 0)
    def _():
      g_issue(0, 0)

    def sloop(i, c):
      even = (i % 2) == 0

      @pl.when((i + 1 < nfull) & (i >= 1))
      def _():
        @pl.when(even)
        def _():
          w_wait(1)
        @pl.when(jnp.logical_not(even))
        def _():
          w_wait(0)

      @pl.when(i + 1 < nfull)
      def _():
        @pl.when(even)
        def _():
          g_issue(i + 1, 1)
        @pl.when(jnp.logical_not(even))
        def _():
          g_issue(i + 1, 0)

      @pl.when(even)
      def _():
        g_wait(0)
        w_issue(i, 0)
      @pl.when(jnp.logical_not(even))
      def _():
        g_wait(1)
        w_issue(i, 1)
      return c

    lax.fori_loop(0, nfull, sloop, 0)

    @pl.when(nfull == 1)
    def _():
      w_wait(0)
    @pl.when(nfull >= 2)
    def _():
      w_wait(0)
      w_wait(1)

    base16 = nfull * CR
    t16 = (n - base16) // L

    def tail16(j, c):
      o = base16 + j * L
      pltpu.async_copy(src_hbm.at[pos_ref[pl.ds(o, L)]], r16, st).wait()
      pltpu.async_copy(r16, out_hbm.at[dst_ref[pl.ds(o, L)]], st).wait()
      return c

    lax.fori_loop(0, t16, tail16, 0)

    rem = n - base16 - t16 * L

    @pl.when((rem > 0) & (n >= L))
    def _():
      o = n - L
      pltpu.async_copy(src_hbm.at[pos_ref[pl.ds(o, L)]], r16, st).wait()
      pltpu.async_copy(r16, out_hbm.at[dst_ref[pl.ds(o, L)]], st).wait()

    @pl.when((rem > 0) & (n < L))
    def _():
      posv = pos_ref[pl.ds(0, L)]
      dstv = dst_ref[pl.ds(0, L)]
      valid = lane < n
      p0 = jnp.max(jnp.where(lane == 0, posv, minv))
      d0 = jnp.max(jnp.where(lane == 0, dstv, minv))
      pltpu.async_copy(src_hbm.at[jnp.where(valid, posv, p0)], r16, st).wait()
      pltpu.async_copy(r16, out_hbm.at[jnp.where(valid, dstv, d0)], st).wait()

  stream(upd_hbm, posf_v, dstf_v, count)
  stream(ast_hbm, dst2_v, dst2_v, count2)


@functools.partial(
    pl.kernel,
    out_type=jax.ShapeDtypeStruct((N_AST, D), jnp.float32),
    mesh=_mesh,
    compiler_params=_sc_params,
    scratch_types=[
        pltpu.VMEM((RPW,), jnp.int32),
        pltpu.VMEM((RPW,), jnp.int32),
        pltpu.VMEM((NPOS,), jnp.int32),
        pltpu.VMEM((NPOS,), jnp.int32),
        pltpu.VMEM((NPOS,), jnp.int32),
        pltpu.VMEM((2, CR, D), jnp.float32),
        pltpu.VMEM((L, D), jnp.float32),
        pltpu.VMEM((CR,), jnp.int32),
        pltpu.VMEM((CR,), jnp.int32),
        pltpu.VMEM((CR,), jnp.int32),
        pltpu.VMEM((CR,), jnp.int32),
        pltpu.SemaphoreType.DMA,
        pltpu.SemaphoreType.DMA,
        pltpu.SemaphoreType.DMA,
        pltpu.SemaphoreType.DMA,
        pltpu.SemaphoreType.DMA,
    ],
)
def _scatter_call(*refs):
  _scatter_body(*refs)


def kernel(previous_ast_nodes_encodings, new_cfg_nodes_encodings,
           key_indices, value_indices, W_g, b_g, W_c, b_c):
  ki = key_indices.astype(jnp.int32)
  vi = value_indices.astype(jnp.int32)
  ki3 = ki.reshape(NW, KPW // C1, C1)
  vi3 = vi.reshape(NW, KPW // C1, C1)
  gp, gu = _gather_call(previous_ast_nodes_encodings, new_cfg_nodes_encodings,
                        ki3, vi3)
  upd = _mlp_call(gp, gu, W_g[:D], W_g[D:], b_g.reshape(1, D),
                  W_c, b_c.reshape(1, D))
  part = _scan_call(ki)
  out = _scatter_call(previous_ast_nodes_encodings, upd, part)
  return out
